# baseline (speedup 1.0000x reference)
import functools

import jax
import jax.numpy as jnp
from jax import lax
from jax.experimental import pallas as pl
from jax.experimental.pallas import tpu as pltpu
from jax.experimental.pallas import tpu_sc as plsc

NC = 2
NS = 16
NW = NC * NS
LANES = 16
CHUNK = 128
STAGE_WORDS = 4000
RING = 8


@functools.cache
def _build(n_fields: int, batch: int, vocab: int):
    b_per_w = batch // NW
    n_chunks = b_per_w // CHUNK
    assert vocab % STAGE_WORDS == 0
    n_stage = vocab // STAGE_WORDS
    stage_per_tile = -(-n_stage // NS)
    mesh = plsc.VectorSubcoreMesh(
        core_axis_name="c", subcore_axis_name="s",
        num_cores=NC, num_subcores=NS,
    )

    @functools.partial(
        pl.kernel,
        out_type=(
            jax.ShapeDtypeStruct((batch,), jnp.float32),
            jax.ShapeDtypeStruct((batch,), jnp.float32),
        ),
        mesh=mesh,
        scratch_types=[
            pltpu.VMEM_SHARED((vocab,), jnp.float32),
            pltpu.VMEM((STAGE_WORDS,), jnp.float32),
            pltpu.VMEM((n_fields, b_per_w), jnp.int32),
            pltpu.VMEM((RING, b_per_w), jnp.float32),
            pltpu.VMEM((b_per_w,), jnp.float32),
            pltpu.VMEM((b_per_w,), jnp.float32),
            pltpu.VMEM((b_per_w,), jnp.float32),
            pltpu.VMEM((LANES,), jnp.float32),
            pltpu.SemaphoreType.DMA,
            pltpu.SemaphoreType.DMA,
        ],
    )
    def k(x_hbm, w_hbm, bias_hbm, lr_hbm, prob_hbm,
          shared_w, stage_v, idx_v, val_v, acc_v, lr_v, prob_v, bias_v,
          sem_g, sem_i):
        sid = lax.axis_index("s")
        wid = sid * NC + lax.axis_index("c")
        base = wid * b_per_w

        idx_copies = [
            pltpu.async_copy(
                x_hbm.at[f, pl.ds(base, b_per_w)], idx_v.at[f], sem_i)
            for f in range(n_fields)
        ]
        pltpu.sync_copy(bias_hbm, bias_v)

        def stage_w(i, carry):
            c = sid * stage_per_tile + i

            @pl.when(c < n_stage)
            def _():
                off = c * STAGE_WORDS
                pltpu.sync_copy(w_hbm.at[pl.ds(off, STAGE_WORDS)], stage_v)
                pltpu.sync_copy(stage_v, shared_w.at[pl.ds(off, STAGE_WORDS)])
            return carry
        lax.fori_loop(0, stage_per_tile, stage_w, 0)
        plsc.subcore_barrier()

        for c in idx_copies:
            c.wait()

        def fire(f):
            slot = lax.rem(f, RING)
            for j in range(n_chunks):
                pltpu.async_copy(
                    shared_w.at[idx_v.at[f, pl.ds(j * CHUNK, CHUNK)]],
                    val_v.at[slot, pl.ds(j * CHUNK, CHUNK)],
                    sem_g,
                )

        def prologue(f, carry):
            fire(f)
            return carry
        lax.fori_loop(0, min(RING, n_fields), prologue, 0)

        bias16 = bias_v[...]

        def init(i, carry):
            acc_v[pl.ds(i * LANES, LANES)] = bias16
            return carry
        lax.fori_loop(0, b_per_w // LANES, init, 0)

        def drain(f, carry):
            slot = lax.rem(f, RING)
            for j in range(n_chunks):
                pltpu.make_async_copy(
                    shared_w.at[idx_v.at[f, pl.ds(j * CHUNK, CHUNK)]],
                    val_v.at[slot, pl.ds(j * CHUNK, CHUNK)],
                    sem_g,
                ).wait()

            def add(i, c2):
                sl = pl.ds(i * LANES, LANES)
                acc_v[sl] = acc_v[sl] + val_v[slot, sl]
                return c2
            lax.fori_loop(0, b_per_w // LANES, add, 0)

            @pl.when(f + RING < n_fields)
            def _():
                fire(f + RING)
            return carry
        lax.fori_loop(0, n_fields, drain, 0)

        def finish(i, carry):
            sl = pl.ds(i * LANES, LANES)
            s = acc_v[sl]
            lr_v[sl] = s
            prob_v[sl] = 1.0 / (1.0 + jnp.exp(-s))
            return carry
        lax.fori_loop(0, b_per_w // LANES, finish, 0)

        pltpu.sync_copy(lr_v, lr_hbm.at[pl.ds(base, b_per_w)])
        pltpu.sync_copy(prob_v, prob_hbm.at[pl.ds(base, b_per_w)])

    return k


def kernel(x, w, b):
    n_fields, batch = x.shape
    vocab = w.shape[0]
    w_flat = w.reshape(-1)
    bias_arr = jnp.broadcast_to(b.astype(jnp.float32), (LANES,))
    lr_flat, prob_flat = _build(n_fields, batch, vocab)(x, w_flat, bias_arr)
    return lr_flat.reshape(batch, 1), prob_flat.reshape(batch, 1)


# speedup vs baseline: 1.0211x; 1.0211x over previous
import functools

import jax
import jax.numpy as jnp
from jax import lax
from jax.experimental import pallas as pl
from jax.experimental.pallas import tpu as pltpu
from jax.experimental.pallas import tpu_sc as plsc

NC = 2
NS = 16
NW = NC * NS
LANES = 16
CHUNK = 128
STAGE_WORDS = 4000
RING = 8


@functools.cache
def _build(n_fields: int, batch: int, vocab: int):
    b_per_w = batch // NW
    n_chunks = b_per_w // CHUNK
    assert vocab % STAGE_WORDS == 0
    n_stage = vocab // STAGE_WORDS
    stage_per_tile = -(-n_stage // NS)
    mesh = plsc.VectorSubcoreMesh(
        core_axis_name="c", subcore_axis_name="s",
        num_cores=NC, num_subcores=NS,
    )

    @functools.partial(
        pl.kernel,
        out_type=(
            jax.ShapeDtypeStruct((batch,), jnp.float32),
            jax.ShapeDtypeStruct((batch,), jnp.float32),
        ),
        mesh=mesh,
        scratch_types=[
            pltpu.VMEM((n_fields, b_per_w), jnp.int32),
            pltpu.VMEM((n_fields, b_per_w), jnp.float32),
            pltpu.VMEM((b_per_w,), jnp.float32),
            pltpu.VMEM((b_per_w,), jnp.float32),
            pltpu.VMEM((b_per_w,), jnp.float32),
            pltpu.VMEM((LANES,), jnp.float32),
            pltpu.SemaphoreType.DMA,
            pltpu.SemaphoreType.DMA,
        ],
    )
    def k(x_hbm, w_hbm, bias_hbm, lr_hbm, prob_hbm,
          idx_v, val_v, acc_v, lr_v, prob_v, bias_v,
          sem_g, sem_i):
        sid = lax.axis_index("s")
        wid = sid * NC + lax.axis_index("c")
        base = wid * b_per_w

        idx_copies = [
            pltpu.async_copy(
                x_hbm.at[f, pl.ds(base, b_per_w)], idx_v.at[f], sem_i)
            for f in range(n_fields)
        ]
        pltpu.sync_copy(bias_hbm, bias_v)

        for c in idx_copies:
            c.wait()

        def fire(f, carry):
            for j in range(n_chunks):
                pltpu.async_copy(
                    w_hbm.at[idx_v.at[f, pl.ds(j * CHUNK, CHUNK)]],
                    val_v.at[f, pl.ds(j * CHUNK, CHUNK)],
                    sem_g,
                )
            return carry
        lax.fori_loop(0, n_fields, fire, 0)

        bias16 = bias_v[...]

        def init(i, carry):
            acc_v[pl.ds(i * LANES, LANES)] = bias16
            return carry
        lax.fori_loop(0, b_per_w // LANES, init, 0)

        def drain(f, carry):
            for j in range(n_chunks):
                pltpu.make_async_copy(
                    w_hbm.at[idx_v.at[f, pl.ds(j * CHUNK, CHUNK)]],
                    val_v.at[f, pl.ds(j * CHUNK, CHUNK)],
                    sem_g,
                ).wait()

            def add(i, c2):
                sl = pl.ds(i * LANES, LANES)
                acc_v[sl] = acc_v[sl] + val_v[f, sl]
                return c2
            lax.fori_loop(0, b_per_w // LANES, add, 0)
            return carry
        lax.fori_loop(0, n_fields, drain, 0)

        def finish(i, carry):
            sl = pl.ds(i * LANES, LANES)
            s = acc_v[sl]
            lr_v[sl] = s
            prob_v[sl] = 1.0 / (1.0 + jnp.exp(-s))
            return carry
        lax.fori_loop(0, b_per_w // LANES, finish, 0)

        pltpu.sync_copy(lr_v, lr_hbm.at[pl.ds(base, b_per_w)])
        pltpu.sync_copy(prob_v, prob_hbm.at[pl.ds(base, b_per_w)])

    return k


def kernel(x, w, b):
    n_fields, batch = x.shape
    vocab = w.shape[0]
    w_flat = w.reshape(-1)
    bias_arr = jnp.broadcast_to(b.astype(jnp.float32), (LANES,))
    lr_flat, prob_flat = _build(n_fields, batch, vocab)(x, w_flat, bias_arr)
    return lr_flat.reshape(batch, 1), prob_flat.reshape(batch, 1)


# speedup vs baseline: 1.0766x; 1.0544x over previous
import functools

import jax
import jax.numpy as jnp
from jax import lax
from jax.experimental import pallas as pl
from jax.experimental.pallas import tpu as pltpu
from jax.experimental.pallas import tpu_sc as plsc

NC = 2
NS = 16
NW = NC * NS
LANES = 16
CHUNK = 128
STAGE_WORDS = 8000


@functools.cache
def _build(n_fields: int, batch: int, vocab: int):
    b_per_w = batch // NW
    assert vocab % STAGE_WORDS == 0
    n_stage = vocab // STAGE_WORDS
    stage_per_tile = -(-n_stage // NS)
    mesh = plsc.VectorSubcoreMesh(
        core_axis_name="c", subcore_axis_name="s",
        num_cores=NC, num_subcores=NS,
    )

    @functools.partial(
        pl.kernel,
        out_type=(
            jax.ShapeDtypeStruct((batch,), jnp.float32),
            jax.ShapeDtypeStruct((batch,), jnp.float32),
        ),
        mesh=mesh,
        scratch_types=[
            pltpu.VMEM_SHARED((vocab,), jnp.float32),
            pltpu.VMEM((STAGE_WORDS,), jnp.float32),
            pltpu.VMEM((STAGE_WORDS,), jnp.float32),
            pltpu.VMEM((n_fields, b_per_w), jnp.int32),
            pltpu.VMEM((n_fields, b_per_w), jnp.float32),
            pltpu.VMEM((b_per_w,), jnp.float32),
            pltpu.VMEM((b_per_w,), jnp.float32),
            pltpu.VMEM((b_per_w,), jnp.float32),
            pltpu.VMEM((LANES,), jnp.float32),
            pltpu.SemaphoreType.DMA,
            pltpu.SemaphoreType.DMA,
            pltpu.SemaphoreType.DMA,
        ],
    )
    def k(x_hbm, w_hbm, bias_hbm, lr_hbm, prob_hbm,
          shared_w, stage_a, stage_b, idx_v, val_v, acc_v, lr_v, prob_v,
          bias_v, sem_g, sem_i, sem_s):
        sid = lax.axis_index("s")
        wid = sid * NC + lax.axis_index("c")
        base = wid * b_per_w

        idx_copies = [
            pltpu.async_copy(
                x_hbm.at[f, pl.ds(base, b_per_w)], idx_v.at[f], sem_i)
            for f in range(n_fields)
        ]
        pltpu.sync_copy(bias_hbm, bias_v)

        bufs = [stage_a, stage_b]

        def hop1(i):
            c = sid * stage_per_tile + i
            return pltpu.async_copy(
                w_hbm.at[pl.ds(c * STAGE_WORDS, STAGE_WORDS)],
                bufs[i % 2], sem_s)

        @pl.when(sid * stage_per_tile < n_stage)
        def _():
            hop1(0)

        for i in range(stage_per_tile):
            c = sid * stage_per_tile + i

            @pl.when(c < n_stage)
            def _(i=i, c=c):
                pltpu.make_async_copy(
                    w_hbm.at[pl.ds(c * STAGE_WORDS, STAGE_WORDS)],
                    bufs[i % 2], sem_s).wait()
                if i + 1 < stage_per_tile:
                    @pl.when(c + 1 < n_stage)
                    def _():
                        hop1(i + 1)
                pltpu.sync_copy(
                    bufs[i % 2],
                    shared_w.at[pl.ds(c * STAGE_WORDS, STAGE_WORDS)])
        plsc.subcore_barrier()

        for c in idx_copies:
            c.wait()

        def fire(f, carry):
            for j in range(b_per_w // CHUNK):
                pltpu.async_copy(
                    shared_w.at[idx_v.at[f, pl.ds(j * CHUNK, CHUNK)]],
                    val_v.at[f, pl.ds(j * CHUNK, CHUNK)],
                    sem_g,
                )
            return carry
        lax.fori_loop(0, n_fields, fire, 0)

        bias16 = bias_v[...]

        def init(i, carry):
            acc_v[pl.ds(i * LANES, LANES)] = bias16
            return carry
        lax.fori_loop(0, b_per_w // LANES, init, 0)

        def drain(f, carry):
            for j in range(b_per_w // CHUNK):
                pltpu.make_async_copy(
                    shared_w.at[idx_v.at[f, pl.ds(j * CHUNK, CHUNK)]],
                    val_v.at[f, pl.ds(j * CHUNK, CHUNK)],
                    sem_g,
                ).wait()

            def add(i, c2):
                sl = pl.ds(i * LANES, LANES)
                acc_v[sl] = acc_v[sl] + val_v[f, sl]
                return c2
            lax.fori_loop(0, b_per_w // LANES, add, 0)
            return carry
        lax.fori_loop(0, n_fields, drain, 0)

        def finish(i, carry):
            sl = pl.ds(i * LANES, LANES)
            s = acc_v[sl]
            lr_v[sl] = s
            prob_v[sl] = 1.0 / (1.0 + jnp.exp(-s))
            return carry
        lax.fori_loop(0, b_per_w // LANES, finish, 0)

        pltpu.sync_copy(lr_v, lr_hbm.at[pl.ds(base, b_per_w)])
        pltpu.sync_copy(prob_v, prob_hbm.at[pl.ds(base, b_per_w)])

    return k


def kernel(x, w, b):
    n_fields, batch = x.shape
    vocab = w.shape[0]
    w_flat = w.reshape(-1)
    bias_arr = jnp.broadcast_to(b.astype(jnp.float32), (LANES,))
    lr_flat, prob_flat = _build(n_fields, batch, vocab)(x, w_flat, bias_arr)
    return lr_flat.reshape(batch, 1), prob_flat.reshape(batch, 1)


# speedup vs baseline: 1.0916x; 1.0139x over previous
import functools

import jax
import jax.numpy as jnp
from jax import lax
from jax.experimental import pallas as pl
from jax.experimental.pallas import tpu as pltpu
from jax.experimental.pallas import tpu_sc as plsc

NC = 2
NS = 16
NW = NC * NS
LANES = 16
CHUNK = 128
STAGE_WORDS = 8000


@functools.cache
def _build(n_fields: int, batch: int, vocab: int):
    b_per_w = batch // NW
    assert vocab % STAGE_WORDS == 0
    n_stage = vocab // STAGE_WORDS
    stage_per_tile = -(-n_stage // NS)
    mesh = plsc.VectorSubcoreMesh(
        core_axis_name="c", subcore_axis_name="s",
        num_cores=NC, num_subcores=NS,
    )

    @functools.partial(
        pl.kernel,
        out_type=(
            jax.ShapeDtypeStruct((batch,), jnp.float32),
            jax.ShapeDtypeStruct((batch,), jnp.float32),
        ),
        mesh=mesh,
        scratch_types=[
            pltpu.VMEM_SHARED((vocab,), jnp.float32),
            pltpu.VMEM((STAGE_WORDS,), jnp.float32),
            pltpu.VMEM((STAGE_WORDS,), jnp.float32),
            pltpu.VMEM((n_fields, b_per_w), jnp.int32),
            pltpu.VMEM((n_fields, b_per_w), jnp.float32),
            pltpu.VMEM((b_per_w,), jnp.float32),
            pltpu.VMEM((b_per_w,), jnp.float32),
            pltpu.VMEM((b_per_w,), jnp.float32),
            pltpu.VMEM((LANES,), jnp.float32),
            pltpu.SemaphoreType.DMA,
            pltpu.SemaphoreType.DMA,
            pltpu.SemaphoreType.DMA,
        ],
    )
    def k(x_hbm, w_hbm, bias_hbm, lr_hbm, prob_hbm,
          shared_w, stage_a, stage_b, idx_v, val_v, acc_v, lr_v, prob_v,
          bias_v, sem_g, sem_i, sem_s):
        sid = lax.axis_index("s")
        wid = sid * NC + lax.axis_index("c")
        base = wid * b_per_w

        idx_copy = pltpu.async_copy(
            x_hbm.at[:, pl.ds(base, b_per_w)], idx_v, sem_i)
        pltpu.sync_copy(bias_hbm, bias_v)

        bufs = [stage_a, stage_b]

        def hop1(i):
            c = sid * stage_per_tile + i
            return pltpu.async_copy(
                w_hbm.at[pl.ds(c * STAGE_WORDS, STAGE_WORDS)],
                bufs[i % 2], sem_s)

        @pl.when(sid * stage_per_tile < n_stage)
        def _():
            hop1(0)

        for i in range(stage_per_tile):
            c = sid * stage_per_tile + i

            @pl.when(c < n_stage)
            def _(i=i, c=c):
                pltpu.make_async_copy(
                    w_hbm.at[pl.ds(c * STAGE_WORDS, STAGE_WORDS)],
                    bufs[i % 2], sem_s).wait()
                if i + 1 < stage_per_tile:
                    @pl.when(c + 1 < n_stage)
                    def _():
                        hop1(i + 1)
                pltpu.sync_copy(
                    bufs[i % 2],
                    shared_w.at[pl.ds(c * STAGE_WORDS, STAGE_WORDS)])
        plsc.subcore_barrier()

        idx_copy.wait()

        def fire(f, carry):
            for j in range(b_per_w // CHUNK):
                pltpu.async_copy(
                    shared_w.at[idx_v.at[f, pl.ds(j * CHUNK, CHUNK)]],
                    val_v.at[f, pl.ds(j * CHUNK, CHUNK)],
                    sem_g,
                )
            return carry
        lax.fori_loop(0, n_fields, fire, 0)

        bias16 = bias_v[...]
        UNROLL = 4

        def wait_field(f):
            for j in range(b_per_w // CHUNK):
                pltpu.make_async_copy(
                    shared_w.at[idx_v.at[f, pl.ds(j * CHUNK, CHUNK)]],
                    val_v.at[f, pl.ds(j * CHUNK, CHUNK)],
                    sem_g,
                ).wait()

        wait_field(0)

        def init(i, carry):
            for u in range(UNROLL):
                sl = pl.ds((i * UNROLL + u) * LANES, LANES)
                acc_v[sl] = bias16 + val_v[0, sl]
            return carry
        lax.fori_loop(0, b_per_w // (LANES * UNROLL), init, 0)

        def drain(f, carry):
            wait_field(f)

            def add(i, c2):
                for u in range(UNROLL):
                    sl = pl.ds((i * UNROLL + u) * LANES, LANES)
                    acc_v[sl] = acc_v[sl] + val_v[f, sl]
                return c2
            lax.fori_loop(0, b_per_w // (LANES * UNROLL), add, 0)
            return carry
        lax.fori_loop(1, n_fields, drain, 0)

        def finish(i, carry):
            for u in range(UNROLL):
                sl = pl.ds((i * UNROLL + u) * LANES, LANES)
                s = acc_v[sl]
                lr_v[sl] = s
                prob_v[sl] = 1.0 / (1.0 + jnp.exp(-s))
            return carry
        lax.fori_loop(0, b_per_w // (LANES * UNROLL), finish, 0)

        pltpu.sync_copy(lr_v, lr_hbm.at[pl.ds(base, b_per_w)])
        pltpu.sync_copy(prob_v, prob_hbm.at[pl.ds(base, b_per_w)])

    return k


def kernel(x, w, b):
    n_fields, batch = x.shape
    vocab = w.shape[0]
    w_flat = w.reshape(-1)
    bias_arr = jnp.broadcast_to(b.astype(jnp.float32), (LANES,))
    lr_flat, prob_flat = _build(n_fields, batch, vocab)(x, w_flat, bias_arr)
    return lr_flat.reshape(batch, 1), prob_flat.reshape(batch, 1)
